# baseline (speedup 1.0000x reference)
import functools

import jax
import jax.numpy as jnp
from jax import lax
from jax.experimental import pallas as pl
from jax.experimental.pallas import tpu as pltpu
from jax.experimental.pallas import tpu_sc as plsc

NC = 2
NS = 16
NW = NC * NS
LANES = 16


NBUF = 4
NIDX = 6


def _sc_aggregate(x, edge_index, edge_weight):
    n, d = x.shape
    e = edge_weight.shape[0]
    epw = e // NW
    k = 80
    ch = epw // k
    blk = 12
    assert e % NW == 0 and epw % k == 0 and k % LANES == 0
    assert ch >= 8 and (ch - 5) % blk == 0
    rpt = (n // NS) & ~7
    tail = n - NS * rpt
    assert tail % 8 == 0 and tail <= k and d % LANES == 0

    mesh = plsc.VectorSubcoreMesh(core_axis_name="c", subcore_axis_name="s")

    @functools.partial(
        pl.kernel,
        out_type=jax.ShapeDtypeStruct((NC, n, d), jnp.float32),
        mesh=mesh,
        scratch_types=(
            [pltpu.VMEM((k,), jnp.int32)] * NIDX
            + [pltpu.VMEM((k,), jnp.int32)] * NIDX
            + [pltpu.VMEM((k,), jnp.float32)] * NIDX
            + [pltpu.VMEM((k, d), jnp.float32)] * NBUF
            + [pltpu.SemaphoreType.DMA] * NIDX
            + [pltpu.SemaphoreType.DMA] * (2 * NBUF)
            + [pltpu.VMEM_SHARED((n, d), jnp.float32)]
        ),
    )
    def sc_kernel(x_hbm, ei_hbm, w_hbm, out_hbm, *refs):
        sbufs = refs[:NIDX]
        dbufs = refs[NIDX:2 * NIDX]
        wbufs = refs[2 * NIDX:3 * NIDX]
        rows = refs[3 * NIDX:3 * NIDX + NBUF]
        isems = refs[3 * NIDX + NBUF:4 * NIDX + NBUF]
        gsems = refs[4 * NIDX + NBUF:4 * NIDX + 2 * NBUF]
        ssems = refs[4 * NIDX + 2 * NBUF:4 * NIDX + 3 * NBUF]
        acc = refs[4 * NIDX + 3 * NBUF]

        cid = lax.axis_index("c")
        sid = lax.axis_index("s")
        wid = sid * NC + cid
        ebase = wid * epw


        def i_start(c, q):
            base = ebase + c * k
            pltpu.async_copy(ei_hbm.at[pl.ds(base, k)], sbufs[q], isems[q])
            pltpu.async_copy(ei_hbm.at[pl.ds(e + base, k)], dbufs[q], isems[q])
            pltpu.async_copy(w_hbm.at[pl.ds(base, k)], wbufs[q], isems[q])

        def i_wait(c, q):
            base = ebase + c * k
            pltpu.make_async_copy(ei_hbm.at[pl.ds(base, k)], sbufs[q],
                                  isems[q]).wait()
            pltpu.make_async_copy(ei_hbm.at[pl.ds(e + base, k)], dbufs[q],
                                  isems[q]).wait()
            pltpu.make_async_copy(w_hbm.at[pl.ds(base, k)], wbufs[q],
                                  isems[q]).wait()

        def g_start(c, p, q):
            pltpu.async_copy(x_hbm.at[sbufs[q]], rows[p], gsems[p])

        def g_wait(c, p, q):
            pltpu.make_async_copy(x_hbm.at[sbufs[q]], rows[p],
                                  gsems[p]).wait()

        def s_start(c, p, q):
            pltpu.async_copy(rows[p], acc.at[dbufs[q]], ssems[p],
                             add=True)

        def s_wait(c, p):
            pltpu.make_async_copy(rows[p], acc.at[pl.ds(0, k)],
                                  ssems[p]).wait()

        def mul(c, p, q):
            rows_v = rows[p]
            w_v = wbufs[q]

            def body(t, carry):
                w16 = w_v[pl.ds(t * LANES, LANES)]
                for i in range(LANES):
                    ws = w16[i]
                    row = t * LANES + i
                    for j in range(d // LANES):
                        sl = pl.ds(j * LANES, LANES)
                        rows_v[row, sl] = rows_v[row, sl] * ws
                return carry

            lax.fori_loop(0, k // LANES, body, 0, unroll=2)

        for c in range(NIDX):
            i_start(c, c)

        zeros16 = jnp.zeros((LANES,), jnp.float32)
        rz = rows[0]

        def zrow(i, carry):
            for j in range(d // LANES):
                rz[i, pl.ds(j * LANES, LANES)] = zeros16
            return carry

        lax.fori_loop(0, k, zrow, 0)
        nfull = rpt // k
        rem = rpt - nfull * k

        def zcopy(t, carry):
            pltpu.sync_copy(rz, acc.at[pl.ds(sid * rpt + t * k, k)])
            return carry

        lax.fori_loop(0, nfull, zcopy, 0)
        if rem:
            pltpu.sync_copy(rz.at[pl.ds(0, rem)],
                            acc.at[pl.ds(sid * rpt + nfull * k, rem)])
        if tail:
            @pl.when(sid == NS - 1)
            def _():
                pltpu.sync_copy(rz.at[pl.ds(0, tail)],
                                acc.at[pl.ds(NS * rpt, tail)])
        plsc.subcore_barrier()

        def step(t):
            p, q = t % NBUF, t % NIDX
            g_wait(t, p, q)
            mul(t, p, q)
            s_start(t, p, q)

        for c in range(3):
            i_wait(c, c)
            g_start(c, c, c)
        step(0)
        i_wait(3, 3)
        g_start(3, 3, 3)
        step(1)
        s_wait(0, 0)
        i_wait(4, 4)
        g_start(4, 0, 4)
        i_start(6, 0)

        def block(g, carry):
            t0 = 2 + blk * g
            for j in range(blk):
                t = t0 + j
                p, q = (2 + j) % NBUF, (2 + j) % NIDX
                p3, q3 = (5 + j) % NBUF, (5 + j) % NIDX
                g_wait(t, p, q)
                mul(t, p, q)
                s_start(t, p, q)
                s_wait(t - 1, (1 + j) % NBUF)

                @pl.when(t <= ch - 6)
                def _():
                    i_start(t + 5, (7 + j) % NIDX)

                @pl.when(t <= ch - 4)
                def _():
                    i_wait(t + 3, q3)
                    g_start(t + 3, p3, q3)
            return carry

        lax.fori_loop(0, (ch - 5) // blk, block, 0)

        for t in range(ch - 3, ch):
            step(t)
            s_wait(t - 1, (t - 1) % NBUF)
            if t + 3 <= ch - 1:
                i_wait(t + 3, (t + 3) % NIDX)
                g_start(t + 3, (t + 3) % NBUF, (t + 3) % NIDX)
        s_wait(ch - 1, (ch - 1) % NBUF)

        plsc.subcore_barrier()
        pltpu.sync_copy(acc.at[pl.ds(sid * rpt, rpt)],
                        out_hbm.at[cid, pl.ds(sid * rpt, rpt)])
        if tail:
            @pl.when(sid == NS - 1)
            def _():
                pltpu.sync_copy(acc.at[pl.ds(NS * rpt, tail)],
                                out_hbm.at[cid, pl.ds(NS * rpt, tail)])

    return sc_kernel(x, edge_index.reshape(-1), edge_weight)


def _tc_finish(partial, W, b2d):
    _, n, d_in = partial.shape
    d_out = W.shape[1]
    rb = 2000
    assert n % rb == 0

    def body(p_ref, w_ref, b_ref, o_ref):
        s = p_ref[0] + p_ref[1]
        h = jnp.dot(s, w_ref[...], preferred_element_type=jnp.float32,
                    precision=lax.Precision.HIGHEST)
        o_ref[...] = jnp.maximum(h + b_ref[...], 0.0)

    return pl.pallas_call(
        body,
        grid=(n // rb,),
        in_specs=[
            pl.BlockSpec((NC, rb, d_in), lambda i: (0, i, 0)),
            pl.BlockSpec((d_in, d_out), lambda i: (0, 0)),
            pl.BlockSpec((1, d_out), lambda i: (0, 0)),
        ],
        out_specs=pl.BlockSpec((rb, d_out), lambda i: (i, 0)),
        out_shape=jax.ShapeDtypeStruct((n, d_out), jnp.float32),
    )(partial, W, b2d)


@jax.jit
def kernel(x, edge_index, edge_weight, W, b):
    partial = _sc_aggregate(x, edge_index, edge_weight)
    return _tc_finish(partial, W, b.reshape(1, -1))


# speedup vs baseline: 1.1034x; 1.1034x over previous
import functools

import jax
import jax.numpy as jnp
from jax import lax
from jax.experimental import pallas as pl
from jax.experimental.pallas import tpu as pltpu
from jax.experimental.pallas import tpu_sc as plsc

NC = 2
NS = 16
NW = NC * NS
LANES = 16


NBUF = 4
NIDX = 6


def _sc_aggregate(x, edge_index, edge_weight):
    n, d = x.shape
    e = edge_weight.shape[0]
    epw = e // NW
    k = 80
    ch = epw // k
    blk = 12
    assert e % NW == 0 and epw % k == 0 and k % LANES == 0
    assert ch >= 8 and (ch - 5) % blk == 0
    rpt = (n // NS) & ~7
    tail = n - NS * rpt
    assert tail % 8 == 0 and tail <= k and d % LANES == 0

    mesh = plsc.VectorSubcoreMesh(core_axis_name="c", subcore_axis_name="s")

    @functools.partial(
        pl.kernel,
        out_type=jax.ShapeDtypeStruct((NC, n, d), jnp.float32),
        mesh=mesh,
        scratch_types=(
            [pltpu.VMEM((k,), jnp.int32)] * NIDX
            + [pltpu.VMEM((k,), jnp.int32)] * NIDX
            + [pltpu.VMEM((k,), jnp.float32)] * NIDX
            + [pltpu.VMEM((k, d), jnp.float32)] * NBUF
            + [pltpu.SemaphoreType.DMA] * NIDX
            + [pltpu.SemaphoreType.DMA] * (2 * NBUF)
            + [pltpu.VMEM_SHARED((n, d), jnp.float32)]
        ),
    )
    def sc_kernel(x_hbm, ei_hbm, w_hbm, out_hbm, *refs):
        sbufs = refs[:NIDX]
        dbufs = refs[NIDX:2 * NIDX]
        wbufs = refs[2 * NIDX:3 * NIDX]
        rows = refs[3 * NIDX:3 * NIDX + NBUF]
        isems = refs[3 * NIDX + NBUF:4 * NIDX + NBUF]
        gsems = refs[4 * NIDX + NBUF:4 * NIDX + 2 * NBUF]
        ssems = refs[4 * NIDX + 2 * NBUF:4 * NIDX + 3 * NBUF]
        acc = refs[4 * NIDX + 3 * NBUF]

        cid = lax.axis_index("c")
        sid = lax.axis_index("s")
        wid = sid * NC + cid
        ebase = wid * epw


        def i_start(c, q):
            base = ebase + c * k
            pltpu.async_copy(ei_hbm.at[pl.ds(base, k)], sbufs[q], isems[q])
            pltpu.async_copy(ei_hbm.at[pl.ds(e + base, k)], dbufs[q], isems[q])
            pltpu.async_copy(w_hbm.at[pl.ds(base, k)], wbufs[q], isems[q])

        def i_wait(c, q):
            base = ebase + c * k
            pltpu.make_async_copy(ei_hbm.at[pl.ds(base, k)], sbufs[q],
                                  isems[q]).wait()
            pltpu.make_async_copy(ei_hbm.at[pl.ds(e + base, k)], dbufs[q],
                                  isems[q]).wait()
            pltpu.make_async_copy(w_hbm.at[pl.ds(base, k)], wbufs[q],
                                  isems[q]).wait()

        def g_start(c, p, q):
            pltpu.async_copy(x_hbm.at[sbufs[q]], rows[p], gsems[p])

        def g_wait(c, p, q):
            pltpu.make_async_copy(x_hbm.at[sbufs[q]], rows[p],
                                  gsems[p]).wait()

        def s_start(c, p, q):
            pltpu.async_copy(rows[p], acc.at[dbufs[q]], ssems[p],
                             add=True)

        def s_wait(c, p):
            pltpu.make_async_copy(rows[p], acc.at[pl.ds(0, k)],
                                  ssems[p]).wait()

        def mul(c, p, q):
            rows_v = rows[p]
            w_v = wbufs[q]

            def body(t, carry):
                w16 = w_v[pl.ds(t * LANES, LANES)]
                for i in range(LANES):
                    ws = w16[i]
                    row = t * LANES + i
                    for j in range(d // LANES):
                        sl = pl.ds(j * LANES, LANES)
                        rows_v[row, sl] = rows_v[row, sl] * ws
                return carry

            lax.fori_loop(0, k // LANES, body, 0)

        for c in range(NIDX):
            i_start(c, c)

        zeros16 = jnp.zeros((LANES,), jnp.float32)
        rz = rows[0]

        def zrow(i, carry):
            for j in range(d // LANES):
                rz[i, pl.ds(j * LANES, LANES)] = zeros16
            return carry

        lax.fori_loop(0, k, zrow, 0)
        nfull = rpt // k
        rem = rpt - nfull * k

        def zcopy(t, carry):
            pltpu.sync_copy(rz, acc.at[pl.ds(sid * rpt + t * k, k)])
            return carry

        lax.fori_loop(0, nfull, zcopy, 0)
        if rem:
            pltpu.sync_copy(rz.at[pl.ds(0, rem)],
                            acc.at[pl.ds(sid * rpt + nfull * k, rem)])
        if tail:
            @pl.when(sid == NS - 1)
            def _():
                pltpu.sync_copy(rz.at[pl.ds(0, tail)],
                                acc.at[pl.ds(NS * rpt, tail)])
        plsc.subcore_barrier()

        def step(t):
            p, q = t % NBUF, t % NIDX
            g_wait(t, p, q)
            mul(t, p, q)
            s_start(t, p, q)

        for c in range(3):
            i_wait(c, c)
            g_start(c, c, c)
        step(0)
        i_wait(3, 3)
        g_start(3, 3, 3)
        step(1)
        s_wait(0, 0)
        i_wait(4, 4)
        g_start(4, 0, 4)
        i_start(6, 0)

        def block(g, carry):
            t0 = 2 + blk * g
            for j in range(blk):
                t = t0 + j
                p, q = (2 + j) % NBUF, (2 + j) % NIDX
                p3, q3 = (5 + j) % NBUF, (5 + j) % NIDX
                g_wait(t, p, q)
                mul(t, p, q)
                s_start(t, p, q)
                s_wait(t - 1, (1 + j) % NBUF)

                @pl.when(t <= ch - 6)
                def _():
                    i_start(t + 5, (7 + j) % NIDX)

                @pl.when(t <= ch - 4)
                def _():
                    i_wait(t + 3, q3)
                    g_start(t + 3, p3, q3)
            return carry

        lax.fori_loop(0, (ch - 5) // blk, block, 0)

        for t in range(ch - 3, ch):
            step(t)
            s_wait(t - 1, (t - 1) % NBUF)
            if t + 3 <= ch - 1:
                i_wait(t + 3, (t + 3) % NIDX)
                g_start(t + 3, (t + 3) % NBUF, (t + 3) % NIDX)
        s_wait(ch - 1, (ch - 1) % NBUF)

        plsc.subcore_barrier()
        pltpu.sync_copy(acc.at[pl.ds(sid * rpt, rpt)],
                        out_hbm.at[cid, pl.ds(sid * rpt, rpt)])
        if tail:
            @pl.when(sid == NS - 1)
            def _():
                pltpu.sync_copy(acc.at[pl.ds(NS * rpt, tail)],
                                out_hbm.at[cid, pl.ds(NS * rpt, tail)])

    return sc_kernel(x, edge_index.reshape(-1), edge_weight)


def _tc_finish(partial, W, b2d):
    _, n, d_in = partial.shape
    d_out = W.shape[1]
    rb = 2000
    assert n % rb == 0

    def body(p_ref, w_ref, b_ref, o_ref):
        s = p_ref[0] + p_ref[1]
        h = jnp.dot(s, w_ref[...], preferred_element_type=jnp.float32,
                    precision=lax.Precision.HIGHEST)
        o_ref[...] = jnp.maximum(h + b_ref[...], 0.0)

    return pl.pallas_call(
        body,
        grid=(n // rb,),
        in_specs=[
            pl.BlockSpec((NC, rb, d_in), lambda i: (0, i, 0)),
            pl.BlockSpec((d_in, d_out), lambda i: (0, 0)),
            pl.BlockSpec((1, d_out), lambda i: (0, 0)),
        ],
        out_specs=pl.BlockSpec((rb, d_out), lambda i: (i, 0)),
        out_shape=jax.ShapeDtypeStruct((n, d_out), jnp.float32),
    )(partial, W, b2d)


@jax.jit
def kernel(x, edge_index, edge_weight, W, b):
    partial = _sc_aggregate(x, edge_index, edge_weight)
    return _tc_finish(partial, W, b.reshape(1, -1))
